# baseline (speedup 1.0000x reference)
import functools

import jax
import jax.numpy as jnp
from jax.experimental import pallas as pl
from jax.experimental.pallas import tpu as pltpu

_INT_MIN = -2147483648


def _keys_kernel(y_ref, pv_ref, keys_ref, *, k_real, kt):
    t = pl.program_id(0)
    yn = y_ref[...]
    pn = pv_ref[...]
    sim = jax.lax.dot_general(yn, pn, (((1,), (1,)), ((), ())),
                              preferred_element_type=jnp.float32)
    xi = jax.lax.bitcast_convert_type(sim, jnp.int32)
    s = jnp.where(xi >= 0, xi, xi ^ jnp.int32(0x7FFFFFFF))
    ki = ~s
    col = t * kt + jax.lax.broadcasted_iota(jnp.int32, sim.shape, 1)
    ki = jnp.where(col < k_real, ki, jnp.int32(0x7FFFFFFF))
    keys_ref[...] = ki


def _select_kernel(keys_ref, gt_ref, out_ref,
                   pu_ref, cless_ref, jst_ref, acc_ref,
                   *, n_groups, kp, b_total, nchunks):
    g = pl.program_id(0)
    p = pl.program_id(1)
    ck = kp // nchunks

    @pl.when(jnp.logical_and(g == 0, p == 0))
    def _init_acc():
        acc_ref[...] = jnp.zeros_like(acc_ref)

    @pl.when(p == 0)
    def _init():
        pu_ref[...] = jnp.zeros_like(pu_ref)
        cless_ref[...] = jnp.zeros_like(cless_ref)

    k = gt_ref[...]

    @pl.when(p < 16)
    def _bit_pass():
        sh2 = 30 - 2 * p
        pu = pu_ref[...]
        cand1_u = pu | jax.lax.shift_left(jnp.int32(1), sh2)
        cand2_u = pu | jax.lax.shift_left(jnp.int32(2), sh2)
        cand3_u = pu | jax.lax.shift_left(jnp.int32(3), sh2)
        cand1_s = cand1_u ^ jnp.int32(_INT_MIN)
        cand2_s = cand2_u ^ jnp.int32(_INT_MIN)
        cand3_s = cand3_u ^ jnp.int32(_INT_MIN)
        c1 = jnp.zeros_like(k)
        c2 = jnp.zeros_like(k)
        c3 = jnp.zeros_like(k)
        for c in range(nchunks):
            kc = keys_ref[:, c * ck:(c + 1) * ck]
            c1 = c1 + jnp.sum((kc < cand1_s).astype(jnp.int32), axis=1,
                              keepdims=True)
            c2 = c2 + jnp.sum((kc < cand2_s).astype(jnp.int32), axis=1,
                              keepdims=True)
            c3 = c3 + jnp.sum((kc < cand3_s).astype(jnp.int32), axis=1,
                              keepdims=True)
        pu_ref[...] = jnp.where(c3 <= k, cand3_u,
                       jnp.where(c2 <= k, cand2_u,
                        jnp.where(c1 <= k, cand1_u, pu)))
        cless_ref[...] = jnp.where(c3 <= k, c3,
                          jnp.where(c2 <= k, c2,
                           jnp.where(c1 <= k, c1, cless_ref[...])))

    @pl.when(p == 16)
    def _resolve():
        v = pu_ref[...] ^ jnp.int32(_INT_MIN)
        r = k - cless_ref[...]
        jst = jnp.full_like(k, kp)
        for c in range(nchunks):
            kc = keys_ref[:, c * ck:(c + 1) * ck]
            lane = c * ck + jax.lax.broadcasted_iota(jnp.int32, kc.shape, 1)
            cand = jnp.where(kc == v, lane, kp)
            jst = jnp.minimum(jst, jnp.min(cand, axis=1, keepdims=True))
        jst_ref[...] = jst

        @pl.when(jnp.any(r > 0))
        def _ties():
            m = jnp.zeros_like(k)
            jst2 = jnp.zeros_like(k)
            for c in range(nchunks):
                kc = keys_ref[:, c * ck:(c + 1) * ck]
                match = (kc == v).astype(jnp.int32)
                lane = jax.lax.broadcasted_iota(jnp.int32, kc.shape, 1)
                incl = match
                sh = 1
                while sh < ck:
                    incl = incl + jnp.where(lane >= sh,
                                            pltpu.roll(incl, sh, 1), 0)
                    sh *= 2
                excl = incl - match
                hit = jnp.logical_and(match == 1, m + excl == r)
                gcol = c * ck + lane
                jst2 = jst2 + jnp.sum(jnp.where(hit, gcol, 0), axis=1,
                                      keepdims=True)
                m = m + jnp.sum(match, axis=1, keepdims=True)
            jst_ref[...] = jst2

        ranks = (jst_ref[...] + 1).astype(jnp.float32)
        acc_ref[...] += jnp.sum(1.0 / ranks, axis=0, keepdims=True)

        @pl.when(g == n_groups - 1)
        def _emit():
            out_ref[...] = acc_ref[...] / b_total


def kernel(y_hat, product_vectors, gt_indices):
    b, d = y_hat.shape
    k_real = product_vectors.shape[0]
    kt = 2048
    n_tiles = -(-k_real // kt)
    kp = n_tiles * kt

    eps = 1e-8
    yn = y_hat / jnp.maximum(jnp.linalg.norm(y_hat, axis=-1, keepdims=True), eps)
    pn = product_vectors / jnp.maximum(
        jnp.linalg.norm(product_vectors, axis=-1, keepdims=True), eps)
    pv = jnp.pad(pn, ((0, kp - k_real), (0, 0)))

    keys = pl.pallas_call(
        functools.partial(_keys_kernel, k_real=k_real, kt=kt),
        grid=(n_tiles,),
        in_specs=[
            pl.BlockSpec((b, d), lambda t: (0, 0)),
            pl.BlockSpec((kt, d), lambda t: (t, 0)),
        ],
        out_specs=pl.BlockSpec((b, kt), lambda t: (0, t)),
        out_shape=jax.ShapeDtypeStruct((b, kp), jnp.int32),
    )(yn, pv)

    gt2 = gt_indices.astype(jnp.int32).reshape(b, 1)

    bg = min(32, b)
    n_groups = b // bg
    nchunks = 8

    mrr = pl.pallas_call(
        functools.partial(_select_kernel, n_groups=n_groups, kp=kp, b_total=b,
                          nchunks=nchunks),
        grid=(n_groups, 17),
        in_specs=[
            pl.BlockSpec((bg, kp), lambda g, p: (g, 0)),
            pl.BlockSpec((bg, 1), lambda g, p: (g, 0)),
        ],
        out_specs=pl.BlockSpec((1, 1), lambda g, p: (0, 0)),
        out_shape=jax.ShapeDtypeStruct((1, 1), jnp.float32),
        scratch_shapes=[
            pltpu.VMEM((bg, 1), jnp.int32),
            pltpu.VMEM((bg, 1), jnp.int32),
            pltpu.VMEM((bg, 1), jnp.int32),
            pltpu.VMEM((1, 1), jnp.float32),
        ],
    )(keys, gt2)

    return mrr[0, 0]


# speedup vs baseline: 1.4466x; 1.4466x over previous
import functools

import jax
import jax.numpy as jnp
from jax.experimental import pallas as pl
from jax.experimental.pallas import tpu as pltpu

_INT_MIN = -2147483648


def _keys_kernel(y_ref, pv_ref, keys_ref, *, k_real, kt):
    t = pl.program_id(0)
    yn = y_ref[...]
    pn = pv_ref[...]
    sim = jax.lax.dot_general(yn, pn, (((1,), (1,)), ((), ())),
                              preferred_element_type=jnp.float32)
    xi = jax.lax.bitcast_convert_type(sim, jnp.int32)
    s = jnp.where(xi >= 0, xi, xi ^ jnp.int32(0x7FFFFFFF))
    ki = ~s
    col = t * kt + jax.lax.broadcasted_iota(jnp.int32, sim.shape, 1)
    ki = jnp.where(col < k_real, ki, jnp.int32(0x7FFFFFFF))
    keys_ref[...] = ki


def _select_kernel(keys_ref, gt_ref, out_ref,
                   pu_ref, cless_ref, jst_ref, acc_ref,
                   *, n_groups, kp, b_total, nchunks):
    g = pl.program_id(0)
    p = pl.program_id(1)
    ck = kp // nchunks

    @pl.when(jnp.logical_and(g == 0, p == 0))
    def _init_acc():
        acc_ref[...] = jnp.zeros_like(acc_ref)

    @pl.when(p == 0)
    def _init():
        pu_ref[...] = jnp.zeros_like(pu_ref)
        cless_ref[...] = jnp.zeros_like(cless_ref)

    k = gt_ref[...]

    @pl.when(p < 32)
    def _bit_pass():
        bit = jax.lax.shift_left(jnp.int32(1), 31 - p)
        cand_u = pu_ref[...] | bit
        cand_s = cand_u ^ jnp.int32(_INT_MIN)
        cnt = jnp.zeros_like(k)
        for c in range(nchunks):
            kc = keys_ref[:, c * ck:(c + 1) * ck]
            cnt = cnt + jnp.sum((kc < cand_s).astype(jnp.int32), axis=1,
                                keepdims=True)
        accept = cnt <= k
        pu_ref[...] = jnp.where(accept, cand_u, pu_ref[...])
        cless_ref[...] = jnp.where(accept, cnt, cless_ref[...])

    @pl.when(p == 32)
    def _resolve():
        v = pu_ref[...] ^ jnp.int32(_INT_MIN)
        r = k - cless_ref[...]
        jst = jnp.full_like(k, kp)
        for c in range(nchunks):
            kc = keys_ref[:, c * ck:(c + 1) * ck]
            lane = c * ck + jax.lax.broadcasted_iota(jnp.int32, kc.shape, 1)
            cand = jnp.where(kc == v, lane, kp)
            jst = jnp.minimum(jst, jnp.min(cand, axis=1, keepdims=True))
        jst_ref[...] = jst

        @pl.when(jnp.any(r > 0))
        def _ties():
            m = jnp.zeros_like(k)
            jst2 = jnp.zeros_like(k)
            for c in range(nchunks):
                kc = keys_ref[:, c * ck:(c + 1) * ck]
                match = (kc == v).astype(jnp.int32)
                lane = jax.lax.broadcasted_iota(jnp.int32, kc.shape, 1)
                incl = match
                sh = 1
                while sh < ck:
                    incl = incl + jnp.where(lane >= sh,
                                            pltpu.roll(incl, sh, 1), 0)
                    sh *= 2
                excl = incl - match
                hit = jnp.logical_and(match == 1, m + excl == r)
                gcol = c * ck + lane
                jst2 = jst2 + jnp.sum(jnp.where(hit, gcol, 0), axis=1,
                                      keepdims=True)
                m = m + jnp.sum(match, axis=1, keepdims=True)
            jst_ref[...] = jst2

        ranks = (jst_ref[...] + 1).astype(jnp.float32)
        acc_ref[...] += jnp.sum(1.0 / ranks, axis=0, keepdims=True)

        @pl.when(g == n_groups - 1)
        def _emit():
            out_ref[...] = acc_ref[...] / b_total


def kernel(y_hat, product_vectors, gt_indices):
    b, d = y_hat.shape
    k_real = product_vectors.shape[0]
    kt = 2048
    n_tiles = -(-k_real // kt)
    kp = n_tiles * kt

    eps = 1e-8
    yn = y_hat / jnp.maximum(jnp.linalg.norm(y_hat, axis=-1, keepdims=True), eps)
    pn = product_vectors / jnp.maximum(
        jnp.linalg.norm(product_vectors, axis=-1, keepdims=True), eps)
    pv = jnp.pad(pn, ((0, kp - k_real), (0, 0)))

    keys = pl.pallas_call(
        functools.partial(_keys_kernel, k_real=k_real, kt=kt),
        grid=(n_tiles,),
        in_specs=[
            pl.BlockSpec((b, d), lambda t: (0, 0)),
            pl.BlockSpec((kt, d), lambda t: (t, 0)),
        ],
        out_specs=pl.BlockSpec((b, kt), lambda t: (0, t)),
        out_shape=jax.ShapeDtypeStruct((b, kp), jnp.int32),
    )(yn, pv)

    gt2 = gt_indices.astype(jnp.int32).reshape(b, 1)

    bg = min(32, b)
    n_groups = b // bg
    nchunks = 8

    mrr = pl.pallas_call(
        functools.partial(_select_kernel, n_groups=n_groups, kp=kp, b_total=b,
                          nchunks=nchunks),
        grid=(n_groups, 33),
        in_specs=[
            pl.BlockSpec((bg, kp), lambda g, p: (g, 0)),
            pl.BlockSpec((bg, 1), lambda g, p: (g, 0)),
        ],
        out_specs=pl.BlockSpec((1, 1), lambda g, p: (0, 0)),
        out_shape=jax.ShapeDtypeStruct((1, 1), jnp.float32),
        scratch_shapes=[
            pltpu.VMEM((bg, 1), jnp.int32),
            pltpu.VMEM((bg, 1), jnp.int32),
            pltpu.VMEM((bg, 1), jnp.int32),
            pltpu.VMEM((1, 1), jnp.float32),
        ],
    )(keys, gt2)

    return mrr[0, 0]
